# baseline (speedup 1.0000x reference)
import functools

import jax
import jax.numpy as jnp
from jax import lax
from jax.experimental import pallas as pl
from jax.experimental.pallas import tpu as pltpu
from jax.experimental.pallas import tpu_sc as plsc

E = 16
D = 1024
NTOK = 4096
BLK = 64
P = NTOK + E * BLK
PB = P // BLK

NC, NS, L = 2, 16, 16
NW = NC * NS

_MESH = plsc.VectorSubcoreMesh(
    core_axis_name="c", subcore_axis_name="s", num_cores=NC, num_subcores=NS)


def _wid():
  return lax.axis_index("s") * NC + lax.axis_index("c")


TOK_T = NTOK // NS
RCH = 64


def _routing_body(gate_hbm, inp_hbm, slots_hbm, bexp_hbm, xbuf_hbm,
                  gate_v, slots4, cur_v, tmp_v, hist_v, bexp_v, rows_v,
                  hist_sh, sem):
  t = lax.axis_index("s")
  c = lax.axis_index("c")
  iota = lax.iota(jnp.int32, L)

  pltpu.sync_copy(gate_hbm.at[pl.ds(t * TOK_T, TOK_T)], gate_v)

  def chunk(i, assign):
    g = gate_v[pl.ds(i * L, L)]
    s, v = plsc.sort_key_val(g, iota + i * L)
    tmp_v[...] = s
    sp = plsc.load_gather(tmp_v, [jnp.maximum(iota - 1, 0)])
    sn = plsc.load_gather(tmp_v, [jnp.minimum(iota + 1, L - 1)])
    boundary = (iota == 0) | (s != sp)
    last = (iota == L - 1) | (s != sn)
    start = plsc.cummax(jnp.where(boundary, iota, 0))
    occ = iota - start
    curg = plsc.load_gather(cur_v, [s])
    slot = curg + occ
    if assign:
      plsc.store_scatter(slots4, [v // RCH, v % RCH], slot)
    plsc.store_scatter(cur_v, [s], slot + 1, mask=last)

  cur_v[...] = jnp.zeros((L,), jnp.int32)

  def pass1(i, carry):
    chunk(i, False)
    return carry
  lax.fori_loop(0, TOK_T // L, pass1, 0)

  pltpu.sync_copy(cur_v, hist_sh.at[pl.ds(t * L, L)])
  plsc.subcore_barrier()
  pltpu.sync_copy(hist_sh, hist_v)

  def acc_hist(tt, carry):
    cnt, pref = carry
    h = hist_v[pl.ds(tt * L, L)]
    return cnt + h, pref + jnp.where(tt < t, h, 0)
  cnt, pref = lax.fori_loop(
      0, NS, acc_hist,
      (jnp.zeros((L,), jnp.int32), jnp.zeros((L,), jnp.int32)))

  padded = ((cnt + BLK - 1) // BLK) * BLK
  inc = plsc.cumsum(padded)
  cur_v[...] = (inc - padded) + pref

  @pl.when((t == 0) & (c == 0))
  def _():
    for vb in range(PB // L):
      mb = (iota + vb * L) * BLK
      acc = jnp.zeros((L,), jnp.int32)
      for e in range(E):
        end_e = jnp.sum(jnp.where(iota == e, inc, 0))
        acc = acc + jnp.where(mb >= end_e, 1, 0)
      bexp_v[pl.ds(vb * L, L)] = jnp.minimum(acc, E - 1)
    pltpu.sync_copy(bexp_v, bexp_hbm)

  def pass2(i, carry):
    chunk(i, True)
    return carry
  lax.fori_loop(0, TOK_T // L, pass2, 0)

  @pl.when(c == 0)
  def _():
    for j in range(TOK_T // RCH):
      pltpu.sync_copy(slots4.at[j],
                      slots_hbm.at[pl.ds(t * TOK_T + j * RCH, RCH)])

  def dispatch(sc):
    def _():
      for h in range(TOK_T // RCH // NC):
        r = sc * (TOK_T // RCH // NC) + h
        pltpu.sync_copy(inp_hbm.at[pl.ds(t * TOK_T + r * RCH, RCH)], rows_v)
        pltpu.async_copy(rows_v, xbuf_hbm.at[slots4.at[r]], sem).wait()
    return _

  for sc in range(NC):
    pl.when(c == sc)(dispatch(sc))


_routing = pl.kernel(
    _routing_body,
    out_type=(
        jax.ShapeDtypeStruct((NTOK,), jnp.int32),
        jax.ShapeDtypeStruct((PB,), jnp.int32),
        jax.ShapeDtypeStruct((P, D), jnp.float32),
    ),
    mesh=_MESH,
    compiler_params=pltpu.CompilerParams(needs_layout_passes=False),
    scratch_types=[
        pltpu.VMEM((TOK_T,), jnp.int32),
        pltpu.VMEM((TOK_T // RCH, RCH), jnp.int32),
        pltpu.VMEM((L,), jnp.int32),
        pltpu.VMEM((L,), jnp.int32),
        pltpu.VMEM((NS * L,), jnp.int32),
        pltpu.VMEM((PB,), jnp.int32),
        pltpu.VMEM((RCH, D), jnp.float32),
        pltpu.VMEM_SHARED((NS * L,), jnp.int32),
        pltpu.SemaphoreType.DMA,
    ],
)


def _gemm_body(bexp_ref, x_ref, w_ref, o_ref):
  o_ref[...] = lax.dot_general(
      x_ref[...], w_ref[0],
      dimension_numbers=(((1,), (1,)), ((), ())),
      preferred_element_type=jnp.float32)


_gemm = pl.pallas_call(
    _gemm_body,
    grid_spec=pltpu.PrefetchScalarGridSpec(
        num_scalar_prefetch=1,
        grid=(PB,),
        in_specs=[
            pl.BlockSpec((BLK, D), lambda i, bexp: (i, 0)),
            pl.BlockSpec((1, D, D), lambda i, bexp: (bexp[i], 0, 0)),
        ],
        out_specs=pl.BlockSpec((BLK, D), lambda i, bexp: (i, 0)),
    ),
    out_shape=jax.ShapeDtypeStruct((P, D), jnp.float32),
)


TOK_W = NTOK // NW
BCH = 64


def _back_body(slots_hbm, ybuf_hbm, out_hbm, idx_v, rows_v, sem):
  base = _wid() * TOK_W

  pltpu.sync_copy(slots_hbm.at[pl.ds(base, TOK_W)], idx_v)

  def step(c, carry):
    pltpu.async_copy(ybuf_hbm.at[idx_v.at[pl.ds(c * BCH, BCH)]], rows_v,
                     sem).wait()
    pltpu.sync_copy(rows_v, out_hbm.at[pl.ds(base + c * BCH, BCH)])
    return carry
  lax.fori_loop(0, TOK_W // BCH, step, 0)


_back = pl.kernel(
    _back_body,
    out_type=jax.ShapeDtypeStruct((NTOK, D), jnp.float32),
    mesh=_MESH,
    scratch_types=[
        pltpu.VMEM((TOK_W,), jnp.int32),
        pltpu.VMEM((BCH, D), jnp.float32),
        pltpu.SemaphoreType.DMA,
    ],
)


@jax.jit
def kernel(inp, gate, weight):
  slots, bexp, x_buf = _routing(gate, inp)
  y_buf = _gemm(bexp, x_buf, weight)
  return _back(slots, y_buf)


# speedup vs baseline: 1.1488x; 1.1488x over previous
import functools

import jax
import jax.numpy as jnp
from jax import lax
from jax.experimental import pallas as pl
from jax.experimental.pallas import tpu as pltpu
from jax.experimental.pallas import tpu_sc as plsc

E = 16
D = 1024
NTOK = 4096
BLK = 128
P = NTOK + E * BLK
PB = P // BLK

NC, NS, L = 2, 16, 16
NW = NC * NS

_MESH = plsc.VectorSubcoreMesh(
    core_axis_name="c", subcore_axis_name="s", num_cores=NC, num_subcores=NS)


def _wid():
  return lax.axis_index("s") * NC + lax.axis_index("c")


TOK_T = NTOK // NS
RCH = 32


def _routing_body(gate_hbm, inp_hbm, slots_hbm, bexp_hbm, xbuf_hbm,
                  gate_v, slots4, cur_v, tmp_v, hist_v, bexp_v, rows_v,
                  rows_b, hist_sh, sem, sem2, sem3):
  t = lax.axis_index("s")
  c = lax.axis_index("c")
  iota = lax.iota(jnp.int32, L)

  pltpu.sync_copy(gate_hbm.at[pl.ds(t * TOK_T, TOK_T)], gate_v)

  def chunk(i, assign):
    g = gate_v[pl.ds(i * L, L)]
    s, v = plsc.sort_key_val(g, iota + i * L)
    tmp_v[...] = s
    sp = plsc.load_gather(tmp_v, [jnp.maximum(iota - 1, 0)])
    sn = plsc.load_gather(tmp_v, [jnp.minimum(iota + 1, L - 1)])
    boundary = (iota == 0) | (s != sp)
    last = (iota == L - 1) | (s != sn)
    start = plsc.cummax(jnp.where(boundary, iota, 0))
    occ = iota - start
    curg = plsc.load_gather(cur_v, [s])
    slot = curg + occ
    if assign:
      plsc.store_scatter(slots4, [v // RCH, v % RCH], slot)
    plsc.store_scatter(cur_v, [s], slot + 1, mask=last)

  cur_v[...] = jnp.zeros((L,), jnp.int32)

  def pass1(i, carry):
    chunk(i, False)
    return carry
  lax.fori_loop(0, TOK_T // L, pass1, 0)

  pltpu.sync_copy(cur_v, hist_sh.at[pl.ds(t * L, L)])
  plsc.subcore_barrier()
  pltpu.sync_copy(hist_sh, hist_v)

  def acc_hist(tt, carry):
    cnt, pref = carry
    h = hist_v[pl.ds(tt * L, L)]
    return cnt + h, pref + jnp.where(tt < t, h, 0)
  cnt, pref = lax.fori_loop(
      0, NS, acc_hist,
      (jnp.zeros((L,), jnp.int32), jnp.zeros((L,), jnp.int32)))

  padded = ((cnt + BLK - 1) // BLK) * BLK
  inc = plsc.cumsum(padded)
  cur_v[...] = (inc - padded) + pref

  @pl.when((t == 0) & (c == 0))
  def _():
    for vb in range(PB // L):
      mb = (iota + vb * L) * BLK
      acc = jnp.zeros((L,), jnp.int32)
      for e in range(E):
        end_e = jnp.sum(jnp.where(iota == e, inc, 0))
        acc = acc + jnp.where(mb >= end_e, 1, 0)
      bexp_v[pl.ds(vb * L, L)] = jnp.minimum(acc, E - 1)
    pltpu.sync_copy(bexp_v, bexp_hbm)

  def pass2(i, carry):
    chunk(i, True)
    return carry
  lax.fori_loop(0, TOK_T // L, pass2, 0)

  @pl.when(c == 0)
  def _():
    for j in range(TOK_T // RCH):
      pltpu.sync_copy(slots4.at[j],
                      slots_hbm.at[pl.ds(t * TOK_T + j * RCH, RCH)])

  def dispatch(sc):
    def _():
      n = TOK_T // RCH // NC
      bufs = (rows_v, rows_b)
      sems = (sem2, sem3)
      pending = None
      for h in range(n):
        r = sc * n + h
        buf, ssem = bufs[h % 2], sems[h % 2]
        pltpu.async_copy(inp_hbm.at[pl.ds(t * TOK_T + r * RCH, RCH)],
                         buf, sem).wait()
        if pending is not None:
          pending.wait()
        pending = pltpu.async_copy(buf, xbuf_hbm.at[slots4.at[r]], ssem)
      pending.wait()
    return _

  for sc in range(NC):
    pl.when(c == sc)(dispatch(sc))


_routing = pl.kernel(
    _routing_body,
    out_type=(
        jax.ShapeDtypeStruct((NTOK,), jnp.int32),
        jax.ShapeDtypeStruct((PB,), jnp.int32),
        jax.ShapeDtypeStruct((P, D), jnp.float32),
    ),
    mesh=_MESH,
    compiler_params=pltpu.CompilerParams(needs_layout_passes=False),
    scratch_types=[
        pltpu.VMEM((TOK_T,), jnp.int32),
        pltpu.VMEM((TOK_T // RCH, RCH), jnp.int32),
        pltpu.VMEM((L,), jnp.int32),
        pltpu.VMEM((L,), jnp.int32),
        pltpu.VMEM((NS * L,), jnp.int32),
        pltpu.VMEM((PB,), jnp.int32),
        pltpu.VMEM((RCH, D), jnp.float32),
        pltpu.VMEM((RCH, D), jnp.float32),
        pltpu.VMEM_SHARED((NS * L,), jnp.int32),
        pltpu.SemaphoreType.DMA,
        pltpu.SemaphoreType.DMA,
        pltpu.SemaphoreType.DMA,
    ],
)


def _gemm_body(bexp_ref, x_ref, w_ref, o_ref):
  o_ref[...] = lax.dot_general(
      x_ref[...], w_ref[0],
      dimension_numbers=(((1,), (1,)), ((), ())),
      preferred_element_type=jnp.float32)


_gemm = pl.pallas_call(
    _gemm_body,
    grid_spec=pltpu.PrefetchScalarGridSpec(
        num_scalar_prefetch=1,
        grid=(PB,),
        in_specs=[
            pl.BlockSpec((BLK, D), lambda i, bexp: (i, 0)),
            pl.BlockSpec((1, D, D), lambda i, bexp: (bexp[i], 0, 0),
                         ),
        ],
        out_specs=pl.BlockSpec((BLK, D), lambda i, bexp: (i, 0)),
    ),
    out_shape=jax.ShapeDtypeStruct((P, D), jnp.float32),
)


TOK_W = NTOK // NW
BCH = 32


def _back_body(slots_hbm, ybuf_hbm, out_hbm, idx_v, rows_v, rows_b, sem,
               sem2, sem3):
  base = _wid() * TOK_W

  pltpu.sync_copy(slots_hbm.at[pl.ds(base, TOK_W)], idx_v)

  bufs = (rows_v, rows_b)
  sems = (sem2, sem3)
  pending = None
  for h in range(TOK_W // BCH):
    buf, ssem = bufs[h % 2], sems[h % 2]
    pltpu.async_copy(ybuf_hbm.at[idx_v.at[pl.ds(h * BCH, BCH)]], buf,
                     sem).wait()
    if pending is not None:
      pending.wait()
    pending = pltpu.async_copy(buf, out_hbm.at[pl.ds(base + h * BCH, BCH)],
                               ssem)
  pending.wait()


_back = pl.kernel(
    _back_body,
    out_type=jax.ShapeDtypeStruct((NTOK, D), jnp.float32),
    mesh=_MESH,
    scratch_types=[
        pltpu.VMEM((TOK_W,), jnp.int32),
        pltpu.VMEM((BCH, D), jnp.float32),
        pltpu.VMEM((BCH, D), jnp.float32),
        pltpu.SemaphoreType.DMA,
        pltpu.SemaphoreType.DMA,
        pltpu.SemaphoreType.DMA,
    ],
)


@jax.jit
def kernel(inp, gate, weight):
  slots, bexp, x_buf = _routing(gate, inp)
  y_buf = _gemm(bexp, x_buf, weight)
  return _back(slots, y_buf)


# speedup vs baseline: 1.1596x; 1.0094x over previous
import functools

import jax
import jax.numpy as jnp
from jax import lax
from jax.experimental import pallas as pl
from jax.experimental.pallas import tpu as pltpu
from jax.experimental.pallas import tpu_sc as plsc

E = 16
D = 1024
NTOK = 4096
BLK = 128
P = NTOK + E * BLK
PB = P // BLK

NC, NS, L = 2, 16, 16
NW = NC * NS

_MESH = plsc.VectorSubcoreMesh(
    core_axis_name="c", subcore_axis_name="s", num_cores=NC, num_subcores=NS)


def _wid():
  return lax.axis_index("s") * NC + lax.axis_index("c")


TOK_T = NTOK // NS
RCH = 64


def _routing_body(gate_hbm, inp_hbm, slots_hbm, bexp_hbm, xbuf_hbm,
                  gate_v, slots4, cur_v, tmp_v, hist_v, bexp_v, rows_v,
                  hist_sh, sem, sem2):
  t = lax.axis_index("s")
  c = lax.axis_index("c")
  iota = lax.iota(jnp.int32, L)

  pltpu.sync_copy(gate_hbm.at[pl.ds(t * TOK_T, TOK_T)], gate_v)

  def chunk(i, assign):
    g = gate_v[pl.ds(i * L, L)]
    s, v = plsc.sort_key_val(g, iota + i * L)
    tmp_v[...] = s
    sp = plsc.load_gather(tmp_v, [jnp.maximum(iota - 1, 0)])
    sn = plsc.load_gather(tmp_v, [jnp.minimum(iota + 1, L - 1)])
    boundary = (iota == 0) | (s != sp)
    last = (iota == L - 1) | (s != sn)
    start = plsc.cummax(jnp.where(boundary, iota, 0))
    occ = iota - start
    curg = plsc.load_gather(cur_v, [s])
    slot = curg + occ
    if assign:
      plsc.store_scatter(slots4, [v // RCH, v % RCH], slot)
    plsc.store_scatter(cur_v, [s], slot + 1, mask=last)

  cur_v[...] = jnp.zeros((L,), jnp.int32)

  def pass1(i, carry):
    chunk(i, False)
    return carry
  lax.fori_loop(0, TOK_T // L, pass1, 0)

  pltpu.sync_copy(cur_v, hist_sh.at[pl.ds(t * L, L)])
  plsc.subcore_barrier()
  pltpu.sync_copy(hist_sh, hist_v)

  def acc_hist(tt, carry):
    cnt, pref = carry
    h = hist_v[pl.ds(tt * L, L)]
    return cnt + h, pref + jnp.where(tt < t, h, 0)
  cnt, pref = lax.fori_loop(
      0, NS, acc_hist,
      (jnp.zeros((L,), jnp.int32), jnp.zeros((L,), jnp.int32)))

  padded = ((cnt + BLK - 1) // BLK) * BLK
  inc = plsc.cumsum(padded)
  cur_v[...] = (inc - padded) + pref

  @pl.when((t == 0) & (c == 0))
  def _():
    for vb in range(PB // L):
      mb = (iota + vb * L) * BLK
      acc = jnp.zeros((L,), jnp.int32)
      for e in range(E):
        end_e = jnp.sum(jnp.where(iota == e, inc, 0))
        acc = acc + jnp.where(mb >= end_e, 1, 0)
      bexp_v[pl.ds(vb * L, L)] = jnp.minimum(acc, E - 1)
    pltpu.sync_copy(bexp_v, bexp_hbm)

  def pass2(i, carry):
    chunk(i, True)
    return carry
  lax.fori_loop(0, TOK_T // L, pass2, 0)

  @pl.when(c == 0)
  def _():
    for j in range(TOK_T // RCH):
      pltpu.sync_copy(slots4.at[j],
                      slots_hbm.at[pl.ds(t * TOK_T + j * RCH, RCH)])

  def dispatch(sc):
    def _():
      n = TOK_T // RCH // NC
      for h in range(n):
        r = sc * n + h
        pltpu.sync_copy(inp_hbm.at[pl.ds(t * TOK_T + r * RCH, RCH)], rows_v)
        pltpu.async_copy(rows_v, xbuf_hbm.at[slots4.at[r]], sem2).wait()
    return _

  for sc in range(NC):
    pl.when(c == sc)(dispatch(sc))


_routing = pl.kernel(
    _routing_body,
    out_type=(
        jax.ShapeDtypeStruct((NTOK,), jnp.int32),
        jax.ShapeDtypeStruct((PB,), jnp.int32),
        jax.ShapeDtypeStruct((P, D), jnp.float32),
    ),
    mesh=_MESH,
    compiler_params=pltpu.CompilerParams(needs_layout_passes=False),
    scratch_types=[
        pltpu.VMEM((TOK_T,), jnp.int32),
        pltpu.VMEM((TOK_T // RCH, RCH), jnp.int32),
        pltpu.VMEM((L,), jnp.int32),
        pltpu.VMEM((L,), jnp.int32),
        pltpu.VMEM((NS * L,), jnp.int32),
        pltpu.VMEM((PB,), jnp.int32),
        pltpu.VMEM((RCH, D), jnp.float32),
        pltpu.VMEM_SHARED((NS * L,), jnp.int32),
        pltpu.SemaphoreType.DMA,
        pltpu.SemaphoreType.DMA,
    ],
)


def _gemm_body(bexp_ref, x_ref, w_ref, o_ref):
  o_ref[...] = lax.dot_general(
      x_ref[...], w_ref[0],
      dimension_numbers=(((1,), (1,)), ((), ())),
      preferred_element_type=jnp.float32)


_gemm = pl.pallas_call(
    _gemm_body,
    grid_spec=pltpu.PrefetchScalarGridSpec(
        num_scalar_prefetch=1,
        grid=(PB,),
        in_specs=[
            pl.BlockSpec((BLK, D), lambda i, bexp: (i, 0)),
            pl.BlockSpec((1, D, D), lambda i, bexp: (bexp[i], 0, 0),
                         ),
        ],
        out_specs=pl.BlockSpec((BLK, D), lambda i, bexp: (i, 0)),
    ),
    out_shape=jax.ShapeDtypeStruct((P, D), jnp.float32),
)


TOK_W = NTOK // NW
BCH = 32


def _back_body(slots_hbm, ybuf_hbm, out_hbm, idx_v, rows_v, rows_b, sem,
               sem2, sem3):
  base = _wid() * TOK_W

  pltpu.sync_copy(slots_hbm.at[pl.ds(base, TOK_W)], idx_v)

  bufs = (rows_v, rows_b)
  sems = (sem2, sem3)
  pending = None
  for h in range(TOK_W // BCH):
    buf, ssem = bufs[h % 2], sems[h % 2]
    pltpu.async_copy(ybuf_hbm.at[idx_v.at[pl.ds(h * BCH, BCH)]], buf,
                     sem).wait()
    if pending is not None:
      pending.wait()
    pending = pltpu.async_copy(buf, out_hbm.at[pl.ds(base + h * BCH, BCH)],
                               ssem)
  pending.wait()


_back = pl.kernel(
    _back_body,
    out_type=jax.ShapeDtypeStruct((NTOK, D), jnp.float32),
    mesh=_MESH,
    scratch_types=[
        pltpu.VMEM((TOK_W,), jnp.int32),
        pltpu.VMEM((BCH, D), jnp.float32),
        pltpu.VMEM((BCH, D), jnp.float32),
        pltpu.SemaphoreType.DMA,
        pltpu.SemaphoreType.DMA,
        pltpu.SemaphoreType.DMA,
    ],
)


@jax.jit
def kernel(inp, gate, weight):
  slots, bexp, x_buf = _routing(gate, inp)
  y_buf = _gemm(bexp, x_buf, weight)
  return _back(slots, y_buf)


# speedup vs baseline: 1.1759x; 1.0140x over previous
import functools

import jax
import jax.numpy as jnp
from jax import lax
from jax.experimental import pallas as pl
from jax.experimental.pallas import tpu as pltpu
from jax.experimental.pallas import tpu_sc as plsc

E = 16
D = 1024
NTOK = 4096
BLK = 128
P = NTOK + E * BLK
PB = P // BLK

NC, NS, L = 2, 16, 16
NW = NC * NS

_MESH = plsc.VectorSubcoreMesh(
    core_axis_name="c", subcore_axis_name="s", num_cores=NC, num_subcores=NS)


def _wid():
  return lax.axis_index("s") * NC + lax.axis_index("c")


TOK_T = NTOK // NS
RCH = 64


def _routing_body(gate_hbm, inp_hbm, slots_hbm, bexp_hbm, xbuf_hbm,
                  gate_v, slots4, cur_v, tmp_v, hist_v, bexp_v, rows_v,
                  hist_sh, sem, sem2):
  t = lax.axis_index("s")
  c = lax.axis_index("c")
  iota = lax.iota(jnp.int32, L)

  pltpu.sync_copy(gate_hbm.at[pl.ds(t * TOK_T, TOK_T)], gate_v)

  def chunk(i, assign):
    g = gate_v[pl.ds(i * L, L)]
    s, v = plsc.sort_key_val(g, iota + i * L)
    tmp_v[...] = s
    sp = plsc.load_gather(tmp_v, [jnp.maximum(iota - 1, 0)])
    sn = plsc.load_gather(tmp_v, [jnp.minimum(iota + 1, L - 1)])
    boundary = (iota == 0) | (s != sp)
    last = (iota == L - 1) | (s != sn)
    start = plsc.cummax(jnp.where(boundary, iota, 0))
    occ = iota - start
    curg = plsc.load_gather(cur_v, [s])
    slot = curg + occ
    if assign:
      plsc.store_scatter(slots4, [v // RCH, v % RCH], slot)
    plsc.store_scatter(cur_v, [s], slot + 1, mask=last)

  cur_v[...] = jnp.zeros((L,), jnp.int32)

  def pass1(i, carry):
    chunk(i, False)
    return carry
  lax.fori_loop(0, TOK_T // L, pass1, 0)

  pltpu.sync_copy(cur_v, hist_sh.at[pl.ds(t * L, L)])
  plsc.subcore_barrier()
  pltpu.sync_copy(hist_sh, hist_v)

  def acc_hist(tt, carry):
    cnt, pref = carry
    h = hist_v[pl.ds(tt * L, L)]
    return cnt + h, pref + jnp.where(tt < t, h, 0)
  cnt, pref = lax.fori_loop(
      0, NS, acc_hist,
      (jnp.zeros((L,), jnp.int32), jnp.zeros((L,), jnp.int32)))

  padded = ((cnt + BLK - 1) // BLK) * BLK
  inc = plsc.cumsum(padded)
  cur_v[...] = (inc - padded) + pref

  @pl.when((t == 0) & (c == 0))
  def _():
    for vb in range(PB // L):
      mb = (iota + vb * L) * BLK
      acc = jnp.zeros((L,), jnp.int32)
      for e in range(E):
        end_e = jnp.sum(jnp.where(iota == e, inc, 0))
        acc = acc + jnp.where(mb >= end_e, 1, 0)
      bexp_v[pl.ds(vb * L, L)] = jnp.minimum(acc, E - 1)
    pltpu.sync_copy(bexp_v, bexp_hbm)

  def pass2(i, carry):
    chunk(i, True)
    return carry
  lax.fori_loop(0, TOK_T // L, pass2, 0)

  @pl.when(c == 0)
  def _():
    for j in range(TOK_T // RCH):
      pltpu.sync_copy(slots4.at[j],
                      slots_hbm.at[pl.ds(t * TOK_T + j * RCH, RCH)])

  def dispatch(sc):
    def _():
      n = TOK_T // RCH // NC
      for h in range(n):
        r = sc * n + h
        pltpu.sync_copy(inp_hbm.at[pl.ds(t * TOK_T + r * RCH, RCH)], rows_v)
        pltpu.async_copy(rows_v, xbuf_hbm.at[slots4.at[r]], sem2).wait()
    return _

  for sc in range(NC):
    pl.when(c == sc)(dispatch(sc))


_routing = pl.kernel(
    _routing_body,
    out_type=(
        jax.ShapeDtypeStruct((NTOK,), jnp.int32),
        jax.ShapeDtypeStruct((PB,), jnp.int32),
        jax.ShapeDtypeStruct((P, D), jnp.float32),
    ),
    mesh=_MESH,
    compiler_params=pltpu.CompilerParams(needs_layout_passes=False),
    scratch_types=[
        pltpu.VMEM((TOK_T,), jnp.int32),
        pltpu.VMEM((TOK_T // RCH, RCH), jnp.int32),
        pltpu.VMEM((L,), jnp.int32),
        pltpu.VMEM((L,), jnp.int32),
        pltpu.VMEM((NS * L,), jnp.int32),
        pltpu.VMEM((PB,), jnp.int32),
        pltpu.VMEM((RCH, D), jnp.float32),
        pltpu.VMEM_SHARED((NS * L,), jnp.int32),
        pltpu.SemaphoreType.DMA,
        pltpu.SemaphoreType.DMA,
    ],
)


def _gemm_body(bexp_ref, x_ref, w_ref, o_ref):
  o_ref[...] = lax.dot_general(
      x_ref[...], w_ref[0],
      dimension_numbers=(((1,), (1,)), ((), ())),
      preferred_element_type=jnp.float32)


_gemm = pl.pallas_call(
    _gemm_body,
    grid_spec=pltpu.PrefetchScalarGridSpec(
        num_scalar_prefetch=1,
        grid=(PB,),
        in_specs=[
            pl.BlockSpec((BLK, D), lambda i, bexp: (i, 0)),
            pl.BlockSpec((1, D, D), lambda i, bexp: (bexp[i], 0, 0),
                         ),
        ],
        out_specs=pl.BlockSpec((BLK, D), lambda i, bexp: (i, 0)),
    ),
    out_shape=jax.ShapeDtypeStruct((P, D), jnp.float32),
)


TOK_W = NTOK // NW
BCH = 64


def _back_body(slots_hbm, ybuf_hbm, out_hbm, idx_v, rows_v, sem):
  base = _wid() * TOK_W

  pltpu.sync_copy(slots_hbm.at[pl.ds(base, TOK_W)], idx_v)

  def step(h, carry):
    pltpu.async_copy(ybuf_hbm.at[idx_v.at[pl.ds(h * BCH, BCH)]], rows_v,
                     sem).wait()
    pltpu.sync_copy(rows_v, out_hbm.at[pl.ds(base + h * BCH, BCH)])
    return carry
  lax.fori_loop(0, TOK_W // BCH, step, 0)


_back = pl.kernel(
    _back_body,
    out_type=jax.ShapeDtypeStruct((NTOK, D), jnp.float32),
    mesh=_MESH,
    scratch_types=[
        pltpu.VMEM((TOK_W,), jnp.int32),
        pltpu.VMEM((BCH, D), jnp.float32),
        pltpu.SemaphoreType.DMA,
    ],
)


@jax.jit
def kernel(inp, gate, weight):
  slots, bexp, x_buf = _routing(gate, inp)
  y_buf = _gemm(bexp, x_buf, weight)
  return _back(slots, y_buf)


# speedup vs baseline: 1.1770x; 1.0010x over previous
import jax
import jax.numpy as jnp
from jax import lax
from jax.experimental import pallas as pl
from jax.experimental.pallas import tpu as pltpu
from jax.experimental.pallas import tpu_sc as plsc

E = 16
D = 1024
NTOK = 4096
BLK = 128
P = NTOK + E * BLK
PB = P // BLK

NC, NS, L = 2, 16, 16
NW = NC * NS

_MESH = plsc.VectorSubcoreMesh(
    core_axis_name="c", subcore_axis_name="s", num_cores=NC, num_subcores=NS)


def _wid():
  return lax.axis_index("s") * NC + lax.axis_index("c")


TOK_T = NTOK // NS
RCH = 64


def _routing_body(gate_hbm, inp_hbm, slots_hbm, bexp_hbm, xbuf_hbm,
                  gate_v, slots4, cur_v, tmp_v, hist_v, bexp_v, rows_v,
                  hist_sh, sem, sem2):
  t = lax.axis_index("s")
  c = lax.axis_index("c")
  iota = lax.iota(jnp.int32, L)

  pltpu.sync_copy(gate_hbm.at[pl.ds(t * TOK_T, TOK_T)], gate_v)

  def chunk(i, assign):
    g = gate_v[pl.ds(i * L, L)]
    s, v = plsc.sort_key_val(g, iota + i * L)
    tmp_v[...] = s
    sp = plsc.load_gather(tmp_v, [jnp.maximum(iota - 1, 0)])
    sn = plsc.load_gather(tmp_v, [jnp.minimum(iota + 1, L - 1)])
    boundary = (iota == 0) | (s != sp)
    last = (iota == L - 1) | (s != sn)
    start = plsc.cummax(jnp.where(boundary, iota, 0))
    occ = iota - start
    curg = plsc.load_gather(cur_v, [s])
    slot = curg + occ
    if assign:
      plsc.store_scatter(slots4, [v // RCH, v % RCH], slot)
    plsc.store_scatter(cur_v, [s], slot + 1, mask=last)

  cur_v[...] = jnp.zeros((L,), jnp.int32)

  def pass1(i, carry):
    chunk(i, False)
    return carry
  lax.fori_loop(0, TOK_T // L, pass1, 0)

  pltpu.sync_copy(cur_v, hist_sh.at[pl.ds(t * L, L)])
  plsc.subcore_barrier()
  pltpu.sync_copy(hist_sh, hist_v)

  def acc_hist(tt, carry):
    cnt, pref = carry
    h = hist_v[pl.ds(tt * L, L)]
    return cnt + h, pref + jnp.where(tt < t, h, 0)
  cnt, pref = lax.fori_loop(
      0, NS, acc_hist,
      (jnp.zeros((L,), jnp.int32), jnp.zeros((L,), jnp.int32)))

  padded = ((cnt + BLK - 1) // BLK) * BLK
  inc = plsc.cumsum(padded)
  cur_v[...] = (inc - padded) + pref

  @pl.when((t == 0) & (c == 0))
  def _():
    for vb in range(PB // L):
      mb = (iota + vb * L) * BLK
      acc = jnp.zeros((L,), jnp.int32)
      for e in range(E):
        end_e = jnp.sum(jnp.where(iota == e, inc, 0))
        acc = acc + jnp.where(mb >= end_e, 1, 0)
      bexp_v[pl.ds(vb * L, L)] = jnp.minimum(acc, E - 1)
    pltpu.sync_copy(bexp_v, bexp_hbm)

  def pass2(i, carry):
    chunk(i, True)
    return carry
  lax.fori_loop(0, TOK_T // L, pass2, 0)

  @pl.when(c == 0)
  def _():
    for j in range(TOK_T // RCH):
      pltpu.sync_copy(slots4.at[j],
                      slots_hbm.at[pl.ds(t * TOK_T + j * RCH, RCH)])

  def dispatch(sc):
    def _():
      n = TOK_T // RCH // NC
      for h in range(n):
        r = sc * n + h
        pltpu.sync_copy(inp_hbm.at[pl.ds(t * TOK_T + r * RCH, RCH)], rows_v)
        pltpu.async_copy(rows_v, xbuf_hbm.at[slots4.at[r]], sem2).wait()
    return _

  for sc in range(NC):
    pl.when(c == sc)(dispatch(sc))


_routing = pl.kernel(
    _routing_body,
    out_type=(
        jax.ShapeDtypeStruct((NTOK,), jnp.int32),
        jax.ShapeDtypeStruct((PB,), jnp.int32),
        jax.ShapeDtypeStruct((P, D), jnp.float32),
    ),
    mesh=_MESH,
    compiler_params=pltpu.CompilerParams(needs_layout_passes=False),
    scratch_types=[
        pltpu.VMEM((TOK_T,), jnp.int32),
        pltpu.VMEM((TOK_T // RCH, RCH), jnp.int32),
        pltpu.VMEM((L,), jnp.int32),
        pltpu.VMEM((L,), jnp.int32),
        pltpu.VMEM((NS * L,), jnp.int32),
        pltpu.VMEM((PB,), jnp.int32),
        pltpu.VMEM((RCH, D), jnp.float32),
        pltpu.VMEM_SHARED((NS * L,), jnp.int32),
        pltpu.SemaphoreType.DMA,
        pltpu.SemaphoreType.DMA,
    ],
)


def _gemm_body(bexp_ref, x_ref, w_ref, o_ref):
  o_ref[...] = lax.dot_general(
      x_ref[...], w_ref[0],
      dimension_numbers=(((1,), (1,)), ((), ())),
      preferred_element_type=jnp.float32)


_gemm = pl.pallas_call(
    _gemm_body,
    grid_spec=pltpu.PrefetchScalarGridSpec(
        num_scalar_prefetch=1,
        grid=(PB,),
        in_specs=[
            pl.BlockSpec((BLK, D), lambda i, bexp: (i, 0)),
            pl.BlockSpec((1, D, D), lambda i, bexp: (bexp[i], 0, 0),
                         ),
        ],
        out_specs=pl.BlockSpec((BLK, D), lambda i, bexp: (i, 0)),
    ),
    out_shape=jax.ShapeDtypeStruct((P, D), jnp.float32),
)


TOK_W = NTOK // NW
BCH = 64


def _back_body(slots_hbm, ybuf_hbm, out_hbm, idx_v, rows_v, sem):
  base = _wid() * TOK_W

  pltpu.sync_copy(slots_hbm.at[pl.ds(base, TOK_W)], idx_v)

  def step(h, carry):
    pltpu.async_copy(ybuf_hbm.at[idx_v.at[pl.ds(h * BCH, BCH)]], rows_v,
                     sem).wait()
    pltpu.sync_copy(rows_v, out_hbm.at[pl.ds(base + h * BCH, BCH)])
    return carry
  lax.fori_loop(0, TOK_W // BCH, step, 0)


_back = pl.kernel(
    _back_body,
    out_type=jax.ShapeDtypeStruct((NTOK, D), jnp.float32),
    mesh=_MESH,
    scratch_types=[
        pltpu.VMEM((TOK_W,), jnp.int32),
        pltpu.VMEM((BCH, D), jnp.float32),
        pltpu.SemaphoreType.DMA,
    ],
)


@jax.jit
def kernel(inp, gate, weight):
  slots, bexp, x_buf = _routing(gate, inp)
  y_buf = _gemm(bexp, x_buf, weight)
  return _back(slots, y_buf)
